# Initial kernel scaffold; baseline (speedup 1.0000x reference)
import functools

import jax
import jax.numpy as jnp
from jax import lax
from jax.experimental import pallas as pl
from jax.experimental.pallas import tpu as pltpu
from jax.experimental.pallas import tpu_sc as plsc

_T = 3
_N = 10000
_E = 320000
_XD = 128
_HD = 128
_ZD = 128

_NC = 2
_NS = 16
_NW = _NC * _NS
_K = 100
_EPW = _E // _NW
_NCH = _EPW // _K
_RPS = _N // _NS


def _spmm_body(sup_ref, src_ref, dst_ref, out_ref,
               zbuf, idx_s, idx_d, rows, agg, sem):
    c = lax.axis_index("c")
    s = lax.axis_index("s")
    w = c * _NS + s

    def _zb(r, carry):
        for k in range(8):
            zbuf[r, pl.ds(16 * k, 16)] = jnp.zeros((16,), jnp.float32)
        return carry
    lax.fori_loop(0, 125, _zb, 0)

    for t in range(_T):
        def _zc(i, carry):
            pltpu.sync_copy(zbuf, agg.at[pl.ds(s * _RPS + i * 125, 125)])
            return carry
        lax.fori_loop(0, 5, _zc, 0)
        plsc.subcore_barrier()

        pltpu.sync_copy(src_ref.at[t, pl.ds(w * _NCH, _NCH)], idx_s)
        pltpu.sync_copy(dst_ref.at[t, pl.ds(w * _NCH, _NCH)], idx_d)

        def _edges(j, carry):
            pltpu.async_copy(sup_ref.at[idx_s.at[j]], rows, sem).wait()
            pltpu.sync_copy(rows, agg.at[idx_d.at[j]], add=True)
            return carry
        lax.fori_loop(0, _NCH, _edges, 0)
        plsc.subcore_barrier()

        pltpu.sync_copy(agg.at[pl.ds(s * _RPS, _RPS)],
                        out_ref.at[t, c, pl.ds(s * _RPS, _RPS)])


@functools.partial(
    pl.kernel,
    out_type=jax.ShapeDtypeStruct((_T, _NC, _N, _HD), jnp.float32),
    mesh=plsc.VectorSubcoreMesh(core_axis_name="c", subcore_axis_name="s"),
    scratch_types=[
        pltpu.VMEM((125, _HD), jnp.float32),
        pltpu.VMEM((_NCH, _K), jnp.int32),
        pltpu.VMEM((_NCH, _K), jnp.int32),
        pltpu.VMEM((_K, _HD), jnp.float32),
        pltpu.VMEM_SHARED((_N, _HD), jnp.float32),
        pltpu.SemaphoreType.DMA,
    ],
)
def _spmm(sup_ref, src_ref, dst_ref, out_ref,
          zbuf, idx_s, idx_d, rows, agg, sem):
    _spmm_body(sup_ref, src_ref, dst_ref, out_ref,
               zbuf, idx_s, idx_d, rows, agg, sem)


_BLKA = 1200


def _phi_gc0_body(x_ref, phiW_ref, phib_ref, gcW0_ref, out_ref):
    h = jnp.maximum(
        jnp.dot(x_ref[...], phiW_ref[...],
                preferred_element_type=jnp.float32) + phib_ref[...], 0.0)
    out_ref[...] = jnp.dot(h, gcW0_ref[...],
                           preferred_element_type=jnp.float32)


_BLKB = 1000


def _gc1_body(agg_ref, gcb0_ref, gcW1_ref, out_ref):
    rep = jnp.maximum(agg_ref[0, 0] + agg_ref[0, 1] + gcb0_ref[...], 0.0)
    out_ref[...] = jnp.dot(rep, gcW1_ref[...],
                           preferred_element_type=jnp.float32)


_BLKC = 1000


def _recur_body(agg_ref, c_ref, gcb1_ref, fuseW_ref, fuseb_ref,
                out00W_ref, out00b_ref, out10W_ref, out10b_ref,
                out01W_ref, out01b_ref, out11W_ref, out11b_ref,
                ps1W_ref, ps1b_ref, bng_ref, bnb_ref, bnm_ref, bnv_ref,
                ps2W_ref, ps2b_ref,
                gruWih_ref, grubih_ref, gruWhh_ref, grubhh_ref,
                y1_ref, y0_ref, z_ref, ps0_ref, ps1o_ref, h_ref):
    h = jnp.zeros((_BLKC, _HD), jnp.float32)
    fuse_Wh = fuseW_ref[...][:_HD, :]
    fuse_Wr = fuseW_ref[...][_HD:, :]
    Wih_z = gruWih_ref[...][:_ZD, :]
    wih_c = gruWih_ref[...][_ZD, :]
    Whh = gruWhh_ref[...]
    inv_std = jax.lax.rsqrt(bnv_ref[...] + 1e-5)
    for t in range(_T):
        rep = jnp.maximum(agg_ref[t, 0] + agg_ref[t, 1] + gcb1_ref[...], 0.0)
        z = jnp.maximum(
            jnp.dot(h, fuse_Wh, preferred_element_type=jnp.float32)
            + jnp.dot(rep, fuse_Wr, preferred_element_type=jnp.float32)
            + fuseb_ref[...], 0.0)
        cf = c_ref[t].astype(jnp.float32)
        gi = (jnp.dot(z, Wih_z, preferred_element_type=jnp.float32)
              + cf[:, None] * wih_c[None, :] + grubih_ref[...])
        gh = (jnp.dot(h, Whh, preferred_element_type=jnp.float32)
              + grubhh_ref[...])
        i_r, i_z, i_n = gi[:, :_HD], gi[:, _HD:2 * _HD], gi[:, 2 * _HD:]
        h_r, h_z, h_n = gh[:, :_HD], gh[:, _HD:2 * _HD], gh[:, 2 * _HD:]
        r = jax.nn.sigmoid(i_r + h_r)
        zg = jax.nn.sigmoid(i_z + h_z)
        ng = jnp.tanh(i_n + r * h_n)
        h = (1.0 - zg) * ng + zg * h

        y00 = jnp.maximum(
            jnp.dot(z, out00W_ref[...], preferred_element_type=jnp.float32)
            + out00b_ref[...], 0.0)
        y10 = jnp.maximum(
            jnp.dot(z, out10W_ref[...], preferred_element_type=jnp.float32)
            + out10b_ref[...], 0.0)
        y0 = jnp.sum(y00 * out01W_ref[...], axis=1) + out01b_ref[0, 0]
        y1 = jnp.sum(y10 * out11W_ref[...], axis=1) + out11b_ref[0, 0]

        p = (jnp.dot(z, ps1W_ref[...], preferred_element_type=jnp.float32)
             + ps1b_ref[...])
        p = (p - bnm_ref[...]) * inv_std * bng_ref[...] + bnb_ref[...]
        p = jax.nn.sigmoid(p)
        q0 = jnp.sum(p * ps2W0_row(ps2W_ref), axis=1) + ps2b_ref[0, 0]
        q1 = jnp.sum(p * ps2W1_row(ps2W_ref), axis=1) + ps2b_ref[0, 1]
        m = jnp.maximum(q0, q1)
        e0 = jnp.exp(q0 - m)
        e1 = jnp.exp(q1 - m)
        denom = e0 + e1
        y1_ref[t] = y1
        y0_ref[t] = y0
        z_ref[t] = z
        ps0_ref[t] = e0 / denom
        ps1o_ref[t] = e1 / denom
    h_ref[...] = h


def ps2W0_row(ps2W_ref):
    return ps2W_ref[...][:, 0][None, :]


def ps2W1_row(ps2W_ref):
    return ps2W_ref[...][:, 1][None, :]


def _full(i):
    return lambda *_: tuple(0 for _ in range(i))


def kernel(X_list, A_list, C_list, phi_W, phi_b, gc_W, gc_b, fuse_W, fuse_b,
           out00_W, out00_b, out10_W, out10_b, out01_W, out01_b,
           out11_W, out11_b, ps1_W, ps1_b, bn_g, bn_b, bn_m, bn_v,
           ps2_W, ps2_b, gru_Wih, gru_bih, gru_Whh, gru_bhh):
    x_all = X_list.reshape(_T * _N, _XD)
    toff = (jnp.arange(_T, dtype=jnp.int32) * _N)[:, None]
    srcs = (A_list[:, 0, :] + toff).reshape(_T, _E // _K, _K)
    dsts = A_list[:, 1, :].reshape(_T, _E // _K, _K)

    support0 = pl.pallas_call(
        _phi_gc0_body,
        grid=(_T * _N // _BLKA,),
        in_specs=[
            pl.BlockSpec((_BLKA, _XD), lambda i: (i, 0)),
            pl.BlockSpec((_XD, _HD), _full(2)),
            pl.BlockSpec((1, _HD), _full(2)),
            pl.BlockSpec((_HD, _HD), _full(2)),
        ],
        out_specs=pl.BlockSpec((_BLKA, _HD), lambda i: (i, 0)),
        out_shape=jax.ShapeDtypeStruct((_T * _N, _HD), jnp.float32),
    )(x_all, phi_W, phi_b.reshape(1, _HD), gc_W[0])

    agg0 = _spmm(support0, srcs, dsts)

    nb = _N // _BLKB
    support1 = pl.pallas_call(
        _gc1_body,
        grid=(_T * nb,),
        in_specs=[
            pl.BlockSpec((1, _NC, _BLKB, _HD),
                         lambda i: (i // nb, 0, i % nb, 0)),
            pl.BlockSpec((1, _HD), _full(2)),
            pl.BlockSpec((_HD, _HD), _full(2)),
        ],
        out_specs=pl.BlockSpec((_BLKB, _HD), lambda i: (i, 0)),
        out_shape=jax.ShapeDtypeStruct((_T * _N, _HD), jnp.float32),
    )(agg0, gc_b[0].reshape(1, _HD), gc_W[1])

    agg1 = _spmm(support1, srcs, dsts)

    nbc = _N // _BLKC
    y1, y0, z_all, ps0, ps1, h = pl.pallas_call(
        _recur_body,
        grid=(nbc,),
        in_specs=[
            pl.BlockSpec((_T, _NC, _BLKC, _HD), lambda i: (0, 0, i, 0)),
            pl.BlockSpec((_T, _BLKC), lambda i: (0, i)),
            pl.BlockSpec((1, _HD), _full(2)),
            pl.BlockSpec((2 * _HD, _ZD), _full(2)),
            pl.BlockSpec((1, _ZD), _full(2)),
            pl.BlockSpec((_ZD, _ZD), _full(2)),
            pl.BlockSpec((1, _ZD), _full(2)),
            pl.BlockSpec((_ZD, _ZD), _full(2)),
            pl.BlockSpec((1, _ZD), _full(2)),
            pl.BlockSpec((1, _ZD), _full(2)),
            pl.BlockSpec((1, 1), _full(2)),
            pl.BlockSpec((1, _ZD), _full(2)),
            pl.BlockSpec((1, 1), _full(2)),
            pl.BlockSpec((_ZD, 100), _full(2)),
            pl.BlockSpec((1, 100), _full(2)),
            pl.BlockSpec((1, 100), _full(2)),
            pl.BlockSpec((1, 100), _full(2)),
            pl.BlockSpec((1, 100), _full(2)),
            pl.BlockSpec((1, 100), _full(2)),
            pl.BlockSpec((100, 2), _full(2)),
            pl.BlockSpec((1, 2), _full(2)),
            pl.BlockSpec((_ZD + 1, 3 * _HD), _full(2)),
            pl.BlockSpec((1, 3 * _HD), _full(2)),
            pl.BlockSpec((_HD, 3 * _HD), _full(2)),
            pl.BlockSpec((1, 3 * _HD), _full(2)),
        ],
        out_specs=[
            pl.BlockSpec((_T, _BLKC), lambda i: (0, i)),
            pl.BlockSpec((_T, _BLKC), lambda i: (0, i)),
            pl.BlockSpec((_T, _BLKC, _ZD), lambda i: (0, i, 0)),
            pl.BlockSpec((_T, _BLKC), lambda i: (0, i)),
            pl.BlockSpec((_T, _BLKC), lambda i: (0, i)),
            pl.BlockSpec((_BLKC, _HD), lambda i: (i, 0)),
        ],
        out_shape=[
            jax.ShapeDtypeStruct((_T, _N), jnp.float32),
            jax.ShapeDtypeStruct((_T, _N), jnp.float32),
            jax.ShapeDtypeStruct((_T, _N, _ZD), jnp.float32),
            jax.ShapeDtypeStruct((_T, _N), jnp.float32),
            jax.ShapeDtypeStruct((_T, _N), jnp.float32),
            jax.ShapeDtypeStruct((_N, _HD), jnp.float32),
        ],
    )(agg1, C_list,
      gc_b[1].reshape(1, _HD), fuse_W, fuse_b.reshape(1, _ZD),
      out00_W, out00_b.reshape(1, _ZD), out10_W, out10_b.reshape(1, _ZD),
      out01_W.reshape(1, _ZD), out01_b.reshape(1, 1),
      out11_W.reshape(1, _ZD), out11_b.reshape(1, 1),
      ps1_W, ps1_b.reshape(1, 100), bn_g.reshape(1, 100),
      bn_b.reshape(1, 100), bn_m.reshape(1, 100), bn_v.reshape(1, 100),
      ps2_W, ps2_b.reshape(1, 2),
      gru_Wih, gru_bih.reshape(1, 3 * _HD), gru_Whh,
      gru_bhh.reshape(1, 3 * _HD))

    ps_hat = jnp.stack([ps0, ps1], axis=-1)
    return (y1, y0, z_all, ps_hat, h)


# speedup vs baseline: 2.6548x; 2.6548x over previous
import functools

import jax
import jax.numpy as jnp
from jax import lax
from jax.experimental import pallas as pl
from jax.experimental.pallas import tpu as pltpu
from jax.experimental.pallas import tpu_sc as plsc

_T = 3
_N = 10000
_E = 320000
_XD = 128
_HD = 128
_ZD = 128

_NC = 1
_NS = 16
_NW = _NC * _NS
_K = 125
_EPW = _E // _NW
_NCH = _EPW // _K
_HCH = _NCH // 2
_NPAD = 10240
_RPS = _NPAD // _NS


def _spmm_body(sup_ref, src_ref, dst_ref, zeros_ref, out_ref,
               idx_s, idx_d, rows, agg, sem):
    c = lax.axis_index("c")
    s = lax.axis_index("s")
    w = c * _NS + s if _NC > 1 else s

    for t in range(_T):
        pltpu.sync_copy(zeros_ref, agg.at[pl.ds(s * _RPS, _RPS)])
        plsc.subcore_barrier()

        for half in range(_NCH // _HCH):
            pltpu.sync_copy(
                src_ref.at[t, pl.ds(w * _NCH + half * _HCH, _HCH)], idx_s)
            pltpu.sync_copy(
                dst_ref.at[t, pl.ds(w * _NCH + half * _HCH, _HCH)], idx_d)

            def _edges(j, carry):
                pltpu.async_copy(sup_ref.at[idx_s.at[j]], rows, sem).wait()
                pltpu.sync_copy(rows, agg.at[idx_d.at[j]], add=True)
                return carry
            lax.fori_loop(0, _HCH, _edges, 0)
        plsc.subcore_barrier()

        @pl.when(s < _NS - 1)
        def _flush_full():
            pltpu.sync_copy(agg.at[pl.ds(s * _RPS, _RPS)],
                            out_ref.at[t, c, pl.ds(s * _RPS, _RPS)])

        @pl.when(s == _NS - 1)
        def _flush_tail():
            pltpu.sync_copy(agg.at[pl.ds((_NS - 1) * _RPS, _N - (_NS - 1) * _RPS)],
                            out_ref.at[t, c, pl.ds((_NS - 1) * _RPS,
                                                   _N - (_NS - 1) * _RPS)])


@functools.lru_cache(maxsize=None)
def _make_spmm():
    return pl.kernel(
        _spmm_body,
        out_type=jax.ShapeDtypeStruct((_T, _NC, _N, _HD), jnp.float32),
        mesh=plsc.VectorSubcoreMesh(core_axis_name="c", subcore_axis_name="s",
                                    num_cores=_NC, num_subcores=_NS),
        scratch_types=[
            pltpu.VMEM((_HCH, _K), jnp.int32),
            pltpu.VMEM((_HCH, _K), jnp.int32),
            pltpu.VMEM((_K, _HD), jnp.float32),
            pltpu.VMEM_SHARED((_NPAD, _HD), jnp.float32),
            pltpu.SemaphoreType.DMA,
        ],
    )


def _spmm(support, srcs, dsts, zeros):
    return _make_spmm()(support, srcs, dsts, zeros)


_BLKA = 1200


def _phi_gc0_body(x_ref, phiW_ref, phib_ref, gcW0_ref, out_ref):
    h = jnp.maximum(
        jnp.dot(x_ref[...], phiW_ref[...],
                preferred_element_type=jnp.float32) + phib_ref[...], 0.0)
    out_ref[...] = jnp.dot(h, gcW0_ref[...],
                           preferred_element_type=jnp.float32)


_BLKB = 1000


def _gc1_body(agg_ref, gcb0_ref, gcW1_ref, out_ref):
    acc = agg_ref[0, 0]
    for c in range(1, _NC):
        acc = acc + agg_ref[0, c]
    rep = jnp.maximum(acc + gcb0_ref[...], 0.0)
    out_ref[...] = jnp.dot(rep, gcW1_ref[...],
                           preferred_element_type=jnp.float32)


_BLKC = 1000


def _recur_body(agg_ref, c_ref, gcb1_ref, fuseW_ref, fuseb_ref,
                out00W_ref, out00b_ref, out10W_ref, out10b_ref,
                out01W_ref, out01b_ref, out11W_ref, out11b_ref,
                ps1W_ref, ps1b_ref, bng_ref, bnb_ref, bnm_ref, bnv_ref,
                ps2W_ref, ps2b_ref,
                gruWih_ref, grubih_ref, gruWhh_ref, grubhh_ref,
                y1_ref, y0_ref, z_ref, ps0_ref, ps1o_ref, h_ref):
    h = jnp.zeros((_BLKC, _HD), jnp.float32)
    fuse_Wh = fuseW_ref[...][:_HD, :]
    fuse_Wr = fuseW_ref[...][_HD:, :]
    Wih_z = gruWih_ref[...][:_ZD, :]
    wih_c = gruWih_ref[...][_ZD, :]
    Whh = gruWhh_ref[...]
    inv_std = jax.lax.rsqrt(bnv_ref[...] + 1e-5)
    for t in range(_T):
        acc = agg_ref[t, 0]
        for cc in range(1, _NC):
            acc = acc + agg_ref[t, cc]
        rep = jnp.maximum(acc + gcb1_ref[...], 0.0)
        z = jnp.maximum(
            jnp.dot(h, fuse_Wh, preferred_element_type=jnp.float32)
            + jnp.dot(rep, fuse_Wr, preferred_element_type=jnp.float32)
            + fuseb_ref[...], 0.0)
        cf = c_ref[0, t].astype(jnp.float32)
        gi = (jnp.dot(z, Wih_z, preferred_element_type=jnp.float32)
              + cf[:, None] * wih_c[None, :] + grubih_ref[...])
        gh = (jnp.dot(h, Whh, preferred_element_type=jnp.float32)
              + grubhh_ref[...])
        i_r, i_z, i_n = gi[:, :_HD], gi[:, _HD:2 * _HD], gi[:, 2 * _HD:]
        h_r, h_z, h_n = gh[:, :_HD], gh[:, _HD:2 * _HD], gh[:, 2 * _HD:]
        r = jax.nn.sigmoid(i_r + h_r)
        zg = jax.nn.sigmoid(i_z + h_z)
        ng = jnp.tanh(i_n + r * h_n)
        h = (1.0 - zg) * ng + zg * h

        y00 = jnp.maximum(
            jnp.dot(z, out00W_ref[...], preferred_element_type=jnp.float32)
            + out00b_ref[...], 0.0)
        y10 = jnp.maximum(
            jnp.dot(z, out10W_ref[...], preferred_element_type=jnp.float32)
            + out10b_ref[...], 0.0)
        y0 = jnp.sum(y00 * out01W_ref[...], axis=1) + out01b_ref[0, 0]
        y1 = jnp.sum(y10 * out11W_ref[...], axis=1) + out11b_ref[0, 0]

        p = (jnp.dot(z, ps1W_ref[...], preferred_element_type=jnp.float32)
             + ps1b_ref[...])
        p = (p - bnm_ref[...]) * inv_std * bng_ref[...] + bnb_ref[...]
        p = jax.nn.sigmoid(p)
        q0 = jnp.sum(p * ps2W0_row(ps2W_ref), axis=1) + ps2b_ref[0, 0]
        q1 = jnp.sum(p * ps2W1_row(ps2W_ref), axis=1) + ps2b_ref[0, 1]
        m = jnp.maximum(q0, q1)
        e0 = jnp.exp(q0 - m)
        e1 = jnp.exp(q1 - m)
        denom = e0 + e1
        y1_ref[0, t] = y1
        y0_ref[0, t] = y0
        z_ref[t] = z
        ps0_ref[0, t] = e0 / denom
        ps1o_ref[0, t] = e1 / denom
    h_ref[...] = h


def ps2W0_row(ps2W_ref):
    return ps2W_ref[...][:, 0][None, :]


def ps2W1_row(ps2W_ref):
    return ps2W_ref[...][:, 1][None, :]


def _full(i):
    return lambda *_: tuple(0 for _ in range(i))


def kernel(X_list, A_list, C_list, phi_W, phi_b, gc_W, gc_b, fuse_W, fuse_b,
           out00_W, out00_b, out10_W, out10_b, out01_W, out01_b,
           out11_W, out11_b, ps1_W, ps1_b, bn_g, bn_b, bn_m, bn_v,
           ps2_W, ps2_b, gru_Wih, gru_bih, gru_Whh, gru_bhh):
    x_all = X_list.reshape(_T * _N, _XD)
    toff = (jnp.arange(_T, dtype=jnp.int32) * _N)[:, None]
    srcs = (A_list[:, 0, :] + toff).reshape(_T, _E // _K, _K)
    dsts = A_list[:, 1, :].reshape(_T, _E // _K, _K)

    support0 = pl.pallas_call(
        _phi_gc0_body,
        grid=(_T * _N // _BLKA,),
        in_specs=[
            pl.BlockSpec((_BLKA, _XD), lambda i: (i, 0)),
            pl.BlockSpec((_XD, _HD), _full(2)),
            pl.BlockSpec((1, _HD), _full(2)),
            pl.BlockSpec((_HD, _HD), _full(2)),
        ],
        out_specs=pl.BlockSpec((_BLKA, _HD), lambda i: (i, 0)),
        out_shape=jax.ShapeDtypeStruct((_T * _N, _HD), jnp.float32),
    )(x_all, phi_W, phi_b.reshape(1, _HD), gc_W[0])

    zeros = jnp.zeros((_RPS, _HD), jnp.float32)
    agg0 = _spmm(support0, srcs, dsts, zeros)

    nb = _N // _BLKB
    support1 = pl.pallas_call(
        _gc1_body,
        grid=(_T * nb,),
        in_specs=[
            pl.BlockSpec((1, _NC, _BLKB, _HD),
                         lambda i: (i // nb, 0, i % nb, 0)),
            pl.BlockSpec((1, _HD), _full(2)),
            pl.BlockSpec((_HD, _HD), _full(2)),
        ],
        out_specs=pl.BlockSpec((_BLKB, _HD), lambda i: (i, 0)),
        out_shape=jax.ShapeDtypeStruct((_T * _N, _HD), jnp.float32),
    )(agg0, gc_b[0].reshape(1, _HD), gc_W[1])

    agg1 = _spmm(support1, srcs, dsts, zeros)

    nbc = _N // _BLKC
    y1, y0, z_all, ps0, ps1, h = pl.pallas_call(
        _recur_body,
        grid=(nbc,),
        in_specs=[
            pl.BlockSpec((_T, _NC, _BLKC, _HD), lambda i: (0, 0, i, 0)),
            pl.BlockSpec((1, _T, _BLKC), lambda i: (i, 0, 0)),
            pl.BlockSpec((1, _HD), _full(2)),
            pl.BlockSpec((2 * _HD, _ZD), _full(2)),
            pl.BlockSpec((1, _ZD), _full(2)),
            pl.BlockSpec((_ZD, _ZD), _full(2)),
            pl.BlockSpec((1, _ZD), _full(2)),
            pl.BlockSpec((_ZD, _ZD), _full(2)),
            pl.BlockSpec((1, _ZD), _full(2)),
            pl.BlockSpec((1, _ZD), _full(2)),
            pl.BlockSpec((1, 1), _full(2)),
            pl.BlockSpec((1, _ZD), _full(2)),
            pl.BlockSpec((1, 1), _full(2)),
            pl.BlockSpec((_ZD, 100), _full(2)),
            pl.BlockSpec((1, 100), _full(2)),
            pl.BlockSpec((1, 100), _full(2)),
            pl.BlockSpec((1, 100), _full(2)),
            pl.BlockSpec((1, 100), _full(2)),
            pl.BlockSpec((1, 100), _full(2)),
            pl.BlockSpec((100, 2), _full(2)),
            pl.BlockSpec((1, 2), _full(2)),
            pl.BlockSpec((_ZD + 1, 3 * _HD), _full(2)),
            pl.BlockSpec((1, 3 * _HD), _full(2)),
            pl.BlockSpec((_HD, 3 * _HD), _full(2)),
            pl.BlockSpec((1, 3 * _HD), _full(2)),
        ],
        out_specs=[
            pl.BlockSpec((1, _T, _BLKC), lambda i: (i, 0, 0)),
            pl.BlockSpec((1, _T, _BLKC), lambda i: (i, 0, 0)),
            pl.BlockSpec((_T, _BLKC, _ZD), lambda i: (0, i, 0)),
            pl.BlockSpec((1, _T, _BLKC), lambda i: (i, 0, 0)),
            pl.BlockSpec((1, _T, _BLKC), lambda i: (i, 0, 0)),
            pl.BlockSpec((_BLKC, _HD), lambda i: (i, 0)),
        ],
        out_shape=[
            jax.ShapeDtypeStruct((nbc, _T, _BLKC), jnp.float32),
            jax.ShapeDtypeStruct((nbc, _T, _BLKC), jnp.float32),
            jax.ShapeDtypeStruct((_T, _N, _ZD), jnp.float32),
            jax.ShapeDtypeStruct((nbc, _T, _BLKC), jnp.float32),
            jax.ShapeDtypeStruct((nbc, _T, _BLKC), jnp.float32),
            jax.ShapeDtypeStruct((_N, _HD), jnp.float32),
        ],
    )(agg1, C_list.reshape(_T, nbc, _BLKC).transpose(1, 0, 2),
      gc_b[1].reshape(1, _HD), fuse_W, fuse_b.reshape(1, _ZD),
      out00_W, out00_b.reshape(1, _ZD), out10_W, out10_b.reshape(1, _ZD),
      out01_W.reshape(1, _ZD), out01_b.reshape(1, 1),
      out11_W.reshape(1, _ZD), out11_b.reshape(1, 1),
      ps1_W, ps1_b.reshape(1, 100), bn_g.reshape(1, 100),
      bn_b.reshape(1, 100), bn_m.reshape(1, 100), bn_v.reshape(1, 100),
      ps2_W, ps2_b.reshape(1, 2),
      gru_Wih, gru_bih.reshape(1, 3 * _HD), gru_Whh,
      gru_bhh.reshape(1, 3 * _HD))

    y1 = y1.transpose(1, 0, 2).reshape(_T, _N)
    y0 = y0.transpose(1, 0, 2).reshape(_T, _N)
    ps0 = ps0.transpose(1, 0, 2).reshape(_T, _N)
    ps1 = ps1.transpose(1, 0, 2).reshape(_T, _N)
    ps_hat = jnp.stack([ps0, ps1], axis=-1)
    return (y1, y0, z_all, ps_hat, h)


# speedup vs baseline: 4.5778x; 1.7244x over previous
import functools

import jax
import jax.numpy as jnp
from jax import lax
from jax.experimental import pallas as pl
from jax.experimental.pallas import tpu as pltpu
from jax.experimental.pallas import tpu_sc as plsc

_T = 3
_N = 10000
_E = 320000
_XD = 128
_HD = 128
_ZD = 128

_NC = 2
_NS = 16
_NW = _NC * _NS
_K = 125
_EPW = _E // _NW
_NCH = _EPW // _K
_HCH = _NCH // 2
_NPAD = 10240
_RPS = _NPAD // _NS


def _spmm_body(sup_ref, src_ref, dst_ref, zeros_ref, out_ref,
               idx_s, idx_d, rows, agg, sem):
    c = lax.axis_index("c")
    s = lax.axis_index("s")
    w = c * _NS + s

    for t in range(_T):
        pltpu.sync_copy(zeros_ref, agg.at[pl.ds(s * _RPS, _RPS)])
        plsc.subcore_barrier()

        for half in range(_NCH // _HCH):
            pltpu.sync_copy(
                src_ref.at[t, pl.ds(w * _NCH + half * _HCH, _HCH)], idx_s)
            pltpu.sync_copy(
                dst_ref.at[t, pl.ds(w * _NCH + half * _HCH, _HCH)], idx_d)

            def _edges(j, carry):
                pltpu.async_copy(sup_ref.at[idx_s.at[j]], rows, sem).wait()
                pltpu.sync_copy(rows, agg.at[idx_d.at[j]], add=True)
                return carry
            lax.fori_loop(0, _HCH, _edges, 0)
        plsc.subcore_barrier()

        @pl.when(s < _NS - 1)
        def _flush_full():
            pltpu.sync_copy(agg.at[pl.ds(s * _RPS, _RPS)],
                            out_ref.at[c, t, pl.ds(s * _RPS, _RPS)])

        @pl.when(s == _NS - 1)
        def _flush_tail():
            pltpu.sync_copy(
                agg.at[pl.ds((_NS - 1) * _RPS, _N - (_NS - 1) * _RPS)],
                out_ref.at[c, t, pl.ds((_NS - 1) * _RPS,
                                       _N - (_NS - 1) * _RPS)])


@functools.lru_cache(maxsize=None)
def _make_spmm():
    return pl.kernel(
        _spmm_body,
        out_type=jax.ShapeDtypeStruct((_NC, _T, _N, _HD), jnp.float32),
        mesh=plsc.VectorSubcoreMesh(core_axis_name="c", subcore_axis_name="s",
                                    num_cores=_NC, num_subcores=_NS),
        scratch_types=[
            pltpu.VMEM((_HCH, _K), jnp.int32),
            pltpu.VMEM((_HCH, _K), jnp.int32),
            pltpu.VMEM((_K, _HD), jnp.float32),
            pltpu.VMEM_SHARED((_NPAD, _HD), jnp.float32),
            pltpu.SemaphoreType.DMA,
        ],
    )


def _spmm(support, srcs, dsts, zeros):
    return _make_spmm()(support, srcs, dsts, zeros)


_BLKA = 1200


def _phi_gc0_body(x_ref, phiW_ref, phib_ref, gcW0_ref, out_ref):
    h = jnp.maximum(
        jnp.dot(x_ref[...], phiW_ref[...],
                preferred_element_type=jnp.float32) + phib_ref[...], 0.0)
    out_ref[...] = jnp.dot(h, gcW0_ref[...],
                           preferred_element_type=jnp.float32)


_BLKB = 1000


def _gc1_body(agg_ref, gcb0_ref, gcW1_ref, out_ref):
    acc = agg_ref[0, 0] + agg_ref[1, 0]
    rep = jnp.maximum(acc + gcb0_ref[...], 0.0)
    out_ref[...] = jnp.dot(rep, gcW1_ref[...],
                           preferred_element_type=jnp.float32)


_BLKC = 1000


def _recur_body(agg_ref, c_ref, gcb1_ref, fuseW_ref, fuseb_ref,
                out00W_ref, out00b_ref, out10W_ref, out10b_ref,
                out01W_ref, out01b_ref, out11W_ref, out11b_ref,
                ps1W_ref, ps1b_ref, bng_ref, bnb_ref, bnm_ref, bnv_ref,
                ps2W_ref, ps2b_ref,
                gruWih_ref, grubih_ref, gruWhh_ref, grubhh_ref,
                y1_ref, y0_ref, z_ref, ps0_ref, ps1o_ref, h_ref):
    h = jnp.zeros((_BLKC, _HD), jnp.float32)
    fuse_Wh = fuseW_ref[...][:_HD, :]
    fuse_Wr = fuseW_ref[...][_HD:, :]
    Wih_z = gruWih_ref[...][:_ZD, :]
    wih_c = gruWih_ref[...][_ZD, :]
    Whh = gruWhh_ref[...]
    inv_std = jax.lax.rsqrt(bnv_ref[...] + 1e-5)
    for t in range(_T):
        acc = agg_ref[0, t] + agg_ref[1, t]
        rep = jnp.maximum(acc + gcb1_ref[...], 0.0)
        z = jnp.maximum(
            jnp.dot(h, fuse_Wh, preferred_element_type=jnp.float32)
            + jnp.dot(rep, fuse_Wr, preferred_element_type=jnp.float32)
            + fuseb_ref[...], 0.0)
        cf = c_ref[0, t].astype(jnp.float32)
        gi = (jnp.dot(z, Wih_z, preferred_element_type=jnp.float32)
              + cf[:, None] * wih_c[None, :] + grubih_ref[...])
        gh = (jnp.dot(h, Whh, preferred_element_type=jnp.float32)
              + grubhh_ref[...])
        i_r, i_z, i_n = gi[:, :_HD], gi[:, _HD:2 * _HD], gi[:, 2 * _HD:]
        h_r, h_z, h_n = gh[:, :_HD], gh[:, _HD:2 * _HD], gh[:, 2 * _HD:]
        r = jax.nn.sigmoid(i_r + h_r)
        zg = jax.nn.sigmoid(i_z + h_z)
        ng = jnp.tanh(i_n + r * h_n)
        h = (1.0 - zg) * ng + zg * h

        y00 = jnp.maximum(
            jnp.dot(z, out00W_ref[...], preferred_element_type=jnp.float32)
            + out00b_ref[...], 0.0)
        y10 = jnp.maximum(
            jnp.dot(z, out10W_ref[...], preferred_element_type=jnp.float32)
            + out10b_ref[...], 0.0)
        y0 = (jnp.dot(y00, out01W_ref[...],
                      preferred_element_type=jnp.float32)[:, 0]
              + out01b_ref[0, 0])
        y1 = (jnp.dot(y10, out11W_ref[...],
                      preferred_element_type=jnp.float32)[:, 0]
              + out11b_ref[0, 0])

        p = (jnp.dot(z, ps1W_ref[...], preferred_element_type=jnp.float32)
             + ps1b_ref[...])
        p = (p - bnm_ref[...]) * inv_std * bng_ref[...] + bnb_ref[...]
        p = jax.nn.sigmoid(p)
        q0 = jnp.sum(p * ps2W0_row(ps2W_ref), axis=1) + ps2b_ref[0, 0]
        q1 = jnp.sum(p * ps2W1_row(ps2W_ref), axis=1) + ps2b_ref[0, 1]
        m = jnp.maximum(q0, q1)
        e0 = jnp.exp(q0 - m)
        e1 = jnp.exp(q1 - m)
        denom = e0 + e1
        y1_ref[0, t] = y1
        y0_ref[0, t] = y0
        z_ref[t] = z
        ps0_ref[0, t] = e0 / denom
        ps1o_ref[0, t] = e1 / denom
    h_ref[...] = h


def _bf(x):
    return x.astype(jnp.bfloat16).astype(jnp.float32)


def ps2W0_row(ps2W_ref):
    return ps2W_ref[...][:, 0][None, :]


def ps2W1_row(ps2W_ref):
    return ps2W_ref[...][:, 1][None, :]


def _full(i):
    return lambda *_: tuple(0 for _ in range(i))


def kernel(X_list, A_list, C_list, phi_W, phi_b, gc_W, gc_b, fuse_W, fuse_b,
           out00_W, out00_b, out10_W, out10_b, out01_W, out01_b,
           out11_W, out11_b, ps1_W, ps1_b, bn_g, bn_b, bn_m, bn_v,
           ps2_W, ps2_b, gru_Wih, gru_bih, gru_Whh, gru_bhh):
    x_all = X_list.reshape(_T * _N, _XD)
    toff = (jnp.arange(_T, dtype=jnp.int32) * _N)[:, None]
    srcs = (A_list[:, 0, :] + toff).reshape(_T, _E // _K, _K)
    dsts = A_list[:, 1, :].reshape(_T, _E // _K, _K)

    support0 = pl.pallas_call(
        _phi_gc0_body,
        grid=(_T * _N // _BLKA,),
        in_specs=[
            pl.BlockSpec((_BLKA, _XD), lambda i: (i, 0)),
            pl.BlockSpec((_XD, _HD), _full(2)),
            pl.BlockSpec((1, _HD), _full(2)),
            pl.BlockSpec((_HD, _HD), _full(2)),
        ],
        out_specs=pl.BlockSpec((_BLKA, _HD), lambda i: (i, 0)),
        out_shape=jax.ShapeDtypeStruct((_T * _N, _HD), jnp.float32),
    )(x_all, phi_W, phi_b.reshape(1, _HD), gc_W[0])

    zeros = jnp.zeros((_RPS, _HD), jnp.float32)
    agg0 = _spmm(support0, srcs, dsts, zeros)

    nb = _N // _BLKB
    support1 = pl.pallas_call(
        _gc1_body,
        grid=(_T * nb,),
        in_specs=[
            pl.BlockSpec((_NC, 1, _BLKB, _HD),
                         lambda i: (0, i // nb, i % nb, 0)),
            pl.BlockSpec((1, _HD), _full(2)),
            pl.BlockSpec((_HD, _HD), _full(2)),
        ],
        out_specs=pl.BlockSpec((_BLKB, _HD), lambda i: (i, 0)),
        out_shape=jax.ShapeDtypeStruct((_T * _N, _HD), jnp.float32),
    )(agg0, gc_b[0].reshape(1, _HD), gc_W[1])

    agg1 = _spmm(support1, srcs, dsts, zeros)

    nbc = _N // _BLKC
    y1, y0, z_all, ps0, ps1, h = pl.pallas_call(
        _recur_body,
        grid=(nbc,),
        in_specs=[
            pl.BlockSpec((_NC, _T, _BLKC, _HD), lambda i: (0, 0, i, 0)),
            pl.BlockSpec((1, _T, _BLKC), lambda i: (i, 0, 0)),
            pl.BlockSpec((1, _HD), _full(2)),
            pl.BlockSpec((2 * _HD, _ZD), _full(2)),
            pl.BlockSpec((1, _ZD), _full(2)),
            pl.BlockSpec((_ZD, _ZD), _full(2)),
            pl.BlockSpec((1, _ZD), _full(2)),
            pl.BlockSpec((_ZD, _ZD), _full(2)),
            pl.BlockSpec((1, _ZD), _full(2)),
            pl.BlockSpec((_ZD, 1), _full(2)),
            pl.BlockSpec((1, 1), _full(2)),
            pl.BlockSpec((_ZD, 1), _full(2)),
            pl.BlockSpec((1, 1), _full(2)),
            pl.BlockSpec((_ZD, 100), _full(2)),
            pl.BlockSpec((1, 100), _full(2)),
            pl.BlockSpec((1, 100), _full(2)),
            pl.BlockSpec((1, 100), _full(2)),
            pl.BlockSpec((1, 100), _full(2)),
            pl.BlockSpec((1, 100), _full(2)),
            pl.BlockSpec((100, 2), _full(2)),
            pl.BlockSpec((1, 2), _full(2)),
            pl.BlockSpec((_ZD + 1, 3 * _HD), _full(2)),
            pl.BlockSpec((1, 3 * _HD), _full(2)),
            pl.BlockSpec((_HD, 3 * _HD), _full(2)),
            pl.BlockSpec((1, 3 * _HD), _full(2)),
        ],
        out_specs=[
            pl.BlockSpec((1, _T, _BLKC), lambda i: (i, 0, 0)),
            pl.BlockSpec((1, _T, _BLKC), lambda i: (i, 0, 0)),
            pl.BlockSpec((_T, _BLKC, _ZD), lambda i: (0, i, 0)),
            pl.BlockSpec((1, _T, _BLKC), lambda i: (i, 0, 0)),
            pl.BlockSpec((1, _T, _BLKC), lambda i: (i, 0, 0)),
            pl.BlockSpec((_BLKC, _HD), lambda i: (i, 0)),
        ],
        out_shape=[
            jax.ShapeDtypeStruct((nbc, _T, _BLKC), jnp.float32),
            jax.ShapeDtypeStruct((nbc, _T, _BLKC), jnp.float32),
            jax.ShapeDtypeStruct((_T, _N, _ZD), jnp.float32),
            jax.ShapeDtypeStruct((nbc, _T, _BLKC), jnp.float32),
            jax.ShapeDtypeStruct((nbc, _T, _BLKC), jnp.float32),
            jax.ShapeDtypeStruct((_N, _HD), jnp.float32),
        ],
    )(agg1, C_list.reshape(_T, nbc, _BLKC).transpose(1, 0, 2),
      gc_b[1].reshape(1, _HD), fuse_W, fuse_b.reshape(1, _ZD),
      out00_W, out00_b.reshape(1, _ZD), out10_W, out10_b.reshape(1, _ZD),
      out01_W, out01_b.reshape(1, 1),
      out11_W, out11_b.reshape(1, 1),
      ps1_W, ps1_b.reshape(1, 100), bn_g.reshape(1, 100),
      bn_b.reshape(1, 100), bn_m.reshape(1, 100), bn_v.reshape(1, 100),
      ps2_W, ps2_b.reshape(1, 2),
      gru_Wih, gru_bih.reshape(1, 3 * _HD), gru_Whh,
      gru_bhh.reshape(1, 3 * _HD))

    y1 = y1.transpose(1, 0, 2).reshape(_T, _N)
    y0 = y0.transpose(1, 0, 2).reshape(_T, _N)
    ps0 = ps0.transpose(1, 0, 2).reshape(_T, _N)
    ps1 = ps1.transpose(1, 0, 2).reshape(_T, _N)
    ps_hat = jnp.stack([ps0, ps1], axis=-1)
    return (y1, y0, z_all, ps_hat, h)


# speedup vs baseline: 6.5731x; 1.4359x over previous
import functools

import jax
import jax.numpy as jnp
from jax import lax
from jax.experimental import pallas as pl
from jax.experimental.pallas import tpu as pltpu
from jax.experimental.pallas import tpu_sc as plsc

_T = 3
_N = 10000
_E = 320000
_XD = 128
_HD = 128
_ZD = 128

_NC = 2
_NS = 16
_NW = _NC * _NS
_K = 125
_EPW = _E // _NW
_NCH = _EPW // _K
_HCH = _NCH // 2
_NPAD = 10240
_RPS = _NPAD // _NS


def _spmm_body(sup_ref, src_ref, dst_ref, zeros_ref, out_ref,
               idx_s, idx_d, rows_a, rows_b, agg, sem_a, sem_b):
    c = lax.axis_index("c")
    s = lax.axis_index("s")
    w = c * _NS + s

    for t in range(_T):
        pltpu.sync_copy(zeros_ref, agg.at[pl.ds(s * _RPS, _RPS)])
        plsc.subcore_barrier()

        for half in range(_NCH // _HCH):
            pltpu.sync_copy(
                src_ref.at[t, pl.ds(w * _NCH + half * _HCH, _HCH)], idx_s)
            pltpu.sync_copy(
                dst_ref.at[t, pl.ds(w * _NCH + half * _HCH, _HCH)], idx_d)

            pltpu.async_copy(sup_ref.at[idx_s.at[0]], rows_a, sem_a)

            def _pair(p, carry):
                j0 = 2 * p
                pltpu.async_copy(sup_ref.at[idx_s.at[j0 + 1]], rows_b, sem_b)
                pltpu.make_async_copy(
                    sup_ref.at[idx_s.at[j0]], rows_a, sem_a).wait()
                pltpu.sync_copy(rows_a, agg.at[idx_d.at[j0]], add=True)

                @pl.when(p < _HCH // 2 - 1)
                def _next_a():
                    pltpu.async_copy(
                        sup_ref.at[idx_s.at[j0 + 2]], rows_a, sem_a)

                pltpu.make_async_copy(
                    sup_ref.at[idx_s.at[j0 + 1]], rows_b, sem_b).wait()
                pltpu.sync_copy(rows_b, agg.at[idx_d.at[j0 + 1]], add=True)
                return carry
            lax.fori_loop(0, _HCH // 2, _pair, 0)
        plsc.subcore_barrier()

        @pl.when(s < _NS - 1)
        def _flush_full():
            pltpu.sync_copy(agg.at[pl.ds(s * _RPS, _RPS)],
                            out_ref.at[c, t, pl.ds(s * _RPS, _RPS)])

        @pl.when(s == _NS - 1)
        def _flush_tail():
            pltpu.sync_copy(
                agg.at[pl.ds((_NS - 1) * _RPS, _N - (_NS - 1) * _RPS)],
                out_ref.at[c, t, pl.ds((_NS - 1) * _RPS,
                                       _N - (_NS - 1) * _RPS)])


@functools.lru_cache(maxsize=None)
def _make_spmm():
    return pl.kernel(
        _spmm_body,
        out_type=jax.ShapeDtypeStruct((_NC, _T, _N, _HD), jnp.float32),
        mesh=plsc.VectorSubcoreMesh(core_axis_name="c", subcore_axis_name="s",
                                    num_cores=_NC, num_subcores=_NS),
        scratch_types=[
            pltpu.VMEM((_HCH, _K), jnp.int32),
            pltpu.VMEM((_HCH, _K), jnp.int32),
            pltpu.VMEM((_K, _HD), jnp.float32),
            pltpu.VMEM((_K, _HD), jnp.float32),
            pltpu.VMEM_SHARED((_NPAD, _HD), jnp.float32),
            pltpu.SemaphoreType.DMA,
            pltpu.SemaphoreType.DMA,
        ],
    )


def _spmm(support, srcs, dsts, zeros):
    return _make_spmm()(support, srcs, dsts, zeros)


_BLKA = 1200


def _phi_gc0_body(x_ref, phiW_ref, phib_ref, gcW0_ref, out_ref):
    h = jnp.maximum(
        jnp.dot(x_ref[...], phiW_ref[...],
                preferred_element_type=jnp.float32) + phib_ref[...], 0.0)
    out_ref[...] = jnp.dot(h, gcW0_ref[...],
                           preferred_element_type=jnp.float32)


_BLKB = 1000


def _gc1_body(agg_ref, gcb0_ref, gcW1_ref, out_ref):
    acc = agg_ref[0, 0] + agg_ref[1, 0]
    rep = jnp.maximum(acc + gcb0_ref[...], 0.0)
    out_ref[...] = jnp.dot(rep, gcW1_ref[...],
                           preferred_element_type=jnp.float32)


_BLKC = 1000


def _recur_body(agg_ref, c_ref, gcb1_ref, fuseW_ref, fuseb_ref,
                out00W_ref, out00b_ref, out10W_ref, out10b_ref,
                out01W_ref, out01b_ref, out11W_ref, out11b_ref,
                ps1W_ref, ps1b_ref, bng_ref, bnb_ref, bnm_ref, bnv_ref,
                ps2W_ref, ps2b_ref,
                gruWih_ref, grubih_ref, gruWhh_ref, grubhh_ref,
                y1_ref, y0_ref, z_ref, ps0_ref, ps1o_ref, h_ref):
    h = jnp.zeros((_BLKC, _HD), jnp.float32)
    fuse_Wh = fuseW_ref[...][:_HD, :]
    fuse_Wr = fuseW_ref[...][_HD:, :]
    Wih_z = gruWih_ref[...][:_ZD, :]
    wih_c = gruWih_ref[...][_ZD, :]
    Whh = gruWhh_ref[...]
    inv_std = jax.lax.rsqrt(bnv_ref[...] + 1e-5)
    for t in range(_T):
        acc = agg_ref[0, t] + agg_ref[1, t]
        rep = jnp.maximum(acc + gcb1_ref[...], 0.0)
        z = jnp.maximum(
            jnp.dot(h, fuse_Wh, preferred_element_type=jnp.float32)
            + jnp.dot(rep, fuse_Wr, preferred_element_type=jnp.float32)
            + fuseb_ref[...], 0.0)
        cf = c_ref[0, t].astype(jnp.float32)
        gi = (jnp.dot(z, Wih_z, preferred_element_type=jnp.float32)
              + cf[:, None] * wih_c[None, :] + grubih_ref[...])
        gh = (jnp.dot(h, Whh, preferred_element_type=jnp.float32)
              + grubhh_ref[...])
        i_r, i_z, i_n = gi[:, :_HD], gi[:, _HD:2 * _HD], gi[:, 2 * _HD:]
        h_r, h_z, h_n = gh[:, :_HD], gh[:, _HD:2 * _HD], gh[:, 2 * _HD:]
        r = jax.nn.sigmoid(i_r + h_r)
        zg = jax.nn.sigmoid(i_z + h_z)
        ng = jnp.tanh(i_n + r * h_n)
        h = (1.0 - zg) * ng + zg * h

        y00 = jnp.maximum(
            jnp.dot(z, out00W_ref[...], preferred_element_type=jnp.float32)
            + out00b_ref[...], 0.0)
        y10 = jnp.maximum(
            jnp.dot(z, out10W_ref[...], preferred_element_type=jnp.float32)
            + out10b_ref[...], 0.0)
        y0 = (jnp.dot(y00, out01W_ref[...],
                      preferred_element_type=jnp.float32)[:, 0]
              + out01b_ref[0, 0])
        y1 = (jnp.dot(y10, out11W_ref[...],
                      preferred_element_type=jnp.float32)[:, 0]
              + out11b_ref[0, 0])

        p = (jnp.dot(z, ps1W_ref[...], preferred_element_type=jnp.float32)
             + ps1b_ref[...])
        p = (p - bnm_ref[...]) * inv_std * bng_ref[...] + bnb_ref[...]
        p = jax.nn.sigmoid(p)
        q0 = jnp.sum(p * ps2W0_row(ps2W_ref), axis=1) + ps2b_ref[0, 0]
        q1 = jnp.sum(p * ps2W1_row(ps2W_ref), axis=1) + ps2b_ref[0, 1]
        m = jnp.maximum(q0, q1)
        e0 = jnp.exp(q0 - m)
        e1 = jnp.exp(q1 - m)
        denom = e0 + e1
        y1_ref[0, t] = y1
        y0_ref[0, t] = y0
        z_ref[t] = z
        ps0_ref[0, t] = e0 / denom
        ps1o_ref[0, t] = e1 / denom
    h_ref[...] = h


def _bf(x):
    return x.astype(jnp.bfloat16).astype(jnp.float32)


def ps2W0_row(ps2W_ref):
    return ps2W_ref[...][:, 0][None, :]


def ps2W1_row(ps2W_ref):
    return ps2W_ref[...][:, 1][None, :]


def _full(i):
    return lambda *_: tuple(0 for _ in range(i))


def kernel(X_list, A_list, C_list, phi_W, phi_b, gc_W, gc_b, fuse_W, fuse_b,
           out00_W, out00_b, out10_W, out10_b, out01_W, out01_b,
           out11_W, out11_b, ps1_W, ps1_b, bn_g, bn_b, bn_m, bn_v,
           ps2_W, ps2_b, gru_Wih, gru_bih, gru_Whh, gru_bhh):
    x_all = X_list.reshape(_T * _N, _XD)
    toff = (jnp.arange(_T, dtype=jnp.int32) * _N)[:, None]
    srcs = (A_list[:, 0, :] + toff).reshape(_T, _E // _K, _K)
    dsts = A_list[:, 1, :].reshape(_T, _E // _K, _K)

    support0 = pl.pallas_call(
        _phi_gc0_body,
        grid=(_T * _N // _BLKA,),
        in_specs=[
            pl.BlockSpec((_BLKA, _XD), lambda i: (i, 0)),
            pl.BlockSpec((_XD, _HD), _full(2)),
            pl.BlockSpec((1, _HD), _full(2)),
            pl.BlockSpec((_HD, _HD), _full(2)),
        ],
        out_specs=pl.BlockSpec((_BLKA, _HD), lambda i: (i, 0)),
        out_shape=jax.ShapeDtypeStruct((_T * _N, _HD), jnp.float32),
    )(x_all, phi_W, phi_b.reshape(1, _HD), gc_W[0])

    zeros = jnp.zeros((_RPS, _HD), jnp.float32)
    agg0 = _spmm(support0, srcs, dsts, zeros)

    nb = _N // _BLKB
    support1 = pl.pallas_call(
        _gc1_body,
        grid=(_T * nb,),
        in_specs=[
            pl.BlockSpec((_NC, 1, _BLKB, _HD),
                         lambda i: (0, i // nb, i % nb, 0)),
            pl.BlockSpec((1, _HD), _full(2)),
            pl.BlockSpec((_HD, _HD), _full(2)),
        ],
        out_specs=pl.BlockSpec((_BLKB, _HD), lambda i: (i, 0)),
        out_shape=jax.ShapeDtypeStruct((_T * _N, _HD), jnp.float32),
    )(agg0, gc_b[0].reshape(1, _HD), gc_W[1])

    agg1 = _spmm(support1, srcs, dsts, zeros)

    nbc = _N // _BLKC
    y1, y0, z_all, ps0, ps1, h = pl.pallas_call(
        _recur_body,
        grid=(nbc,),
        in_specs=[
            pl.BlockSpec((_NC, _T, _BLKC, _HD), lambda i: (0, 0, i, 0)),
            pl.BlockSpec((1, _T, _BLKC), lambda i: (i, 0, 0)),
            pl.BlockSpec((1, _HD), _full(2)),
            pl.BlockSpec((2 * _HD, _ZD), _full(2)),
            pl.BlockSpec((1, _ZD), _full(2)),
            pl.BlockSpec((_ZD, _ZD), _full(2)),
            pl.BlockSpec((1, _ZD), _full(2)),
            pl.BlockSpec((_ZD, _ZD), _full(2)),
            pl.BlockSpec((1, _ZD), _full(2)),
            pl.BlockSpec((_ZD, 1), _full(2)),
            pl.BlockSpec((1, 1), _full(2)),
            pl.BlockSpec((_ZD, 1), _full(2)),
            pl.BlockSpec((1, 1), _full(2)),
            pl.BlockSpec((_ZD, 100), _full(2)),
            pl.BlockSpec((1, 100), _full(2)),
            pl.BlockSpec((1, 100), _full(2)),
            pl.BlockSpec((1, 100), _full(2)),
            pl.BlockSpec((1, 100), _full(2)),
            pl.BlockSpec((1, 100), _full(2)),
            pl.BlockSpec((100, 2), _full(2)),
            pl.BlockSpec((1, 2), _full(2)),
            pl.BlockSpec((_ZD + 1, 3 * _HD), _full(2)),
            pl.BlockSpec((1, 3 * _HD), _full(2)),
            pl.BlockSpec((_HD, 3 * _HD), _full(2)),
            pl.BlockSpec((1, 3 * _HD), _full(2)),
        ],
        out_specs=[
            pl.BlockSpec((1, _T, _BLKC), lambda i: (i, 0, 0)),
            pl.BlockSpec((1, _T, _BLKC), lambda i: (i, 0, 0)),
            pl.BlockSpec((_T, _BLKC, _ZD), lambda i: (0, i, 0)),
            pl.BlockSpec((1, _T, _BLKC), lambda i: (i, 0, 0)),
            pl.BlockSpec((1, _T, _BLKC), lambda i: (i, 0, 0)),
            pl.BlockSpec((_BLKC, _HD), lambda i: (i, 0)),
        ],
        out_shape=[
            jax.ShapeDtypeStruct((nbc, _T, _BLKC), jnp.float32),
            jax.ShapeDtypeStruct((nbc, _T, _BLKC), jnp.float32),
            jax.ShapeDtypeStruct((_T, _N, _ZD), jnp.float32),
            jax.ShapeDtypeStruct((nbc, _T, _BLKC), jnp.float32),
            jax.ShapeDtypeStruct((nbc, _T, _BLKC), jnp.float32),
            jax.ShapeDtypeStruct((_N, _HD), jnp.float32),
        ],
    )(agg1, C_list.reshape(_T, nbc, _BLKC).transpose(1, 0, 2),
      gc_b[1].reshape(1, _HD), fuse_W, fuse_b.reshape(1, _ZD),
      out00_W, out00_b.reshape(1, _ZD), out10_W, out10_b.reshape(1, _ZD),
      out01_W, out01_b.reshape(1, 1),
      out11_W, out11_b.reshape(1, 1),
      ps1_W, ps1_b.reshape(1, 100), bn_g.reshape(1, 100),
      bn_b.reshape(1, 100), bn_m.reshape(1, 100), bn_v.reshape(1, 100),
      ps2_W, ps2_b.reshape(1, 2),
      gru_Wih, gru_bih.reshape(1, 3 * _HD), gru_Whh,
      gru_bhh.reshape(1, 3 * _HD))

    y1 = y1.transpose(1, 0, 2).reshape(_T, _N)
    y0 = y0.transpose(1, 0, 2).reshape(_T, _N)
    ps0 = ps0.transpose(1, 0, 2).reshape(_T, _N)
    ps1 = ps1.transpose(1, 0, 2).reshape(_T, _N)
    ps_hat = jnp.stack([ps0, ps1], axis=-1)
    return (y1, y0, z_all, ps_hat, h)
